# baseline (speedup 1.0000x reference)
import functools

import jax
import jax.numpy as jnp
from jax import lax
from jax.experimental import pallas as pl
from jax.experimental.pallas import tpu as pltpu
from jax.experimental.pallas import tpu_sc as plsc

N = 10000
E = 320000
F = 128
H = 64
G = 128
K = 2
C = 2

NC = 2
NS = 16
NW = NC * NS
CH = 128
NBUF = 4

NPAD = 10240
ND = 10000
BRANGE = NPAD // NW
CAP = 92 * CH
NWCAP = NW * CAP
NCH_S = CAP // CH
NCH_G = 2 * CAP // CH
RPS = NPAD // NS
W1C = 8

RB = 2048
EB = 4096

_MESH = plsc.VectorSubcoreMesh(
    core_axis_name="c", subcore_axis_name="s", num_cores=NC, num_subcores=NS)
_SC_PARAMS = pltpu.CompilerParams(use_tc_tiling_on_sc=False)

f32 = jnp.float32
i32 = jnp.int32


def _make_spmm(W):
    def body(table, src_i, dst_i, zrows, out, acc, idx_s, idx_d, rows, *sems):
        cid = lax.axis_index("c")
        sid = lax.axis_index("s")
        wid = sid * NC + cid
        base = sid * RPS
        pltpu.sync_copy(zrows, rows.at[0])
        for j in range(RPS // CH):
            pltpu.sync_copy(rows.at[0], acc.at[pl.ds(base + j * CH, CH)])
        pltpu.sync_copy(src_i.at[wid], idx_s)
        pltpu.sync_copy(dst_i.at[wid], idx_d)
        plsc.subcore_barrier()

        def grp(g, carry):
            cps = [pltpu.async_copy(table.at[idx_s.at[g * NBUF + b]], rows.at[b],
                                    sems[b])
                   for b in range(NBUF)]
            for b in range(NBUF):
                cps[b].wait()
                pltpu.sync_copy(rows.at[b], acc.at[idx_d.at[g * NBUF + b]], add=True)
            return carry

        lax.fori_loop(0, NCH_S // NBUF, grp, 0)
        plsc.subcore_barrier()
        for j in range(RPS // CH):
            r = base + j * CH
            pltpu.sync_copy(acc.at[pl.ds(r, CH)], rows.at[0])
            pltpu.sync_copy(rows.at[0], out.at[pl.ds(cid * NPAD + r, CH)])

    return pl.kernel(
        body,
        out_type=jax.ShapeDtypeStruct((NC * NPAD, W), f32),
        mesh=_MESH,
        scratch_types=[
            pltpu.VMEM_SHARED((NPAD, W), f32),
            pltpu.VMEM((NCH_S, CH), i32),
            pltpu.VMEM((NCH_S, CH), i32),
            pltpu.VMEM((NBUF, CH, W), f32),
        ] + [pltpu.SemaphoreType.DMA] * NBUF,
        compiler_params=_SC_PARAMS,
    )


_sc_spmm_call = _make_spmm(H)


def _sc_spmm(table, src_r, dst_r, zrows):
    out = _sc_spmm_call(table, src_r, dst_r, zrows)
    return out[:NPAD], out[NPAD:]


def _sc_gather_body(table, idx_i, out, idx_v, rows, *sems):
    cid = lax.axis_index("c")
    sid = lax.axis_index("s")
    wid = sid * NC + cid
    pltpu.sync_copy(idx_i.at[wid], idx_v)

    def grp(g, carry):
        cps = [pltpu.async_copy(table.at[idx_v.at[g * NBUF + b]], rows.at[b],
                                sems[b])
               for b in range(NBUF)]
        for b in range(NBUF):
            cps[b].wait()
            r = wid * (NCH_G * CH) + (g * NBUF + b) * CH
            pltpu.sync_copy(rows.at[b], out.at[pl.ds(r, CH)])
        return carry

    lax.fori_loop(0, NCH_G // NBUF, grp, 0)


_sc_gather_call = pl.kernel(
    _sc_gather_body,
    out_type=jax.ShapeDtypeStruct((2 * NWCAP, H), f32),
    mesh=_MESH,
    scratch_types=[
        pltpu.VMEM((NCH_G, CH), i32),
        pltpu.VMEM((NBUF, CH, H), f32),
    ] + [pltpu.SemaphoreType.DMA] * NBUF,
    compiler_params=_SC_PARAMS,
)


def _sc_count_body(idx_i, ones_i, zrows, out, acc, idx_v, ones_v, zbuf):
    cid = lax.axis_index("c")
    sid = lax.axis_index("s")
    wid = sid * NC + cid
    base = sid * RPS
    pltpu.sync_copy(zrows, zbuf)
    for j in range(RPS // CH):
        pltpu.sync_copy(zbuf, acc.at[pl.ds(base + j * CH, CH)])
    pltpu.sync_copy(ones_i, ones_v)
    pltpu.sync_copy(idx_i.at[wid], idx_v)
    plsc.subcore_barrier()

    def step(ch, carry):
        pltpu.sync_copy(ones_v, acc.at[idx_v.at[ch]], add=True)
        return carry

    lax.fori_loop(0, NCH_G, step, 0)
    plsc.subcore_barrier()
    for j in range(RPS // CH):
        r = base + j * CH
        pltpu.sync_copy(acc.at[pl.ds(r, CH)], zbuf)
        pltpu.sync_copy(zbuf, out.at[pl.ds(cid * NPAD + r, CH)])


_sc_count_call = pl.kernel(
    _sc_count_body,
    out_type=jax.ShapeDtypeStruct((NC * NPAD, W1C), f32),
    mesh=_MESH,
    scratch_types=[
        pltpu.VMEM_SHARED((NPAD, W1C), f32),
        pltpu.VMEM((NCH_G, CH), i32),
        pltpu.VMEM((CH, W1C), f32),
        pltpu.VMEM((CH, W1C), f32),
    ],
    compiler_params=_SC_PARAMS,
)


def _vm(shape):
    return pl.BlockSpec(shape, lambda *_: (0,) * len(shape))


def _rows(bshape):
    return pl.BlockSpec(bshape, lambda i: (i,) + (0,) * (len(bshape) - 1))


def _dot(a, b):
    return jnp.dot(a, b, preferred_element_type=f32)


def _t0_body(x_ref, w_ref, o_ref):
    o_ref[...] = _dot(x_ref[...], w_ref[...])


def _t0(x_pad, wcat):
    return pl.pallas_call(
        _t0_body,
        grid=(NPAD // RB,),
        in_specs=[_rows((RB, F)), _vm((F, 2 * H))],
        out_specs=_rows((RB, 2 * H)),
        out_shape=jax.ShapeDtypeStruct((NPAD, 2 * H), f32),
    )(x_pad, wcat)


def _mlp(p, a_a, a_b, eps, b1, w2, b2):
    t = jnp.maximum((1.0 + eps[0, 0]) * p + a_a + a_b + b1[...], 0.0)
    return jnp.maximum(_dot(t, w2[...]) + b2[...], 0.0)


def _gin_mid_body(p_ref, aa_ref, ab_ref, eps, b1, w2, b2, w1n, o_ref):
    h = _mlp(p_ref[...], aa_ref[...], ab_ref[...], eps, b1, w2, b2)
    ridx = pl.program_id(0) * RB + lax.broadcasted_iota(i32, (RB, 1), 0)
    o_ref[...] = jnp.where(ridx < N, _dot(h, w1n[...]), 0.0)


def _gin_mid(p, a_a, a_b, lay, w1n):
    return pl.pallas_call(
        _gin_mid_body,
        grid=(NPAD // RB,),
        in_specs=[_rows((RB, H))] * 3
        + [_vm((1, 1)), _vm((1, H)), _vm((H, H)), _vm((1, H)), _vm((H, H))],
        out_specs=_rows((RB, H)),
        out_shape=jax.ShapeDtypeStruct((NPAD, H), f32),
    )(p, a_a, a_b, lay["eps"].reshape(1, 1), lay["b1"][None], lay["W2"],
      lay["b2"][None], w1n)


def _gin_ref_body(h_ref, aa_ref, ab_ref, eps, b1, w1, b2, w2, o_ref):
    z = (1.0 + eps[0, 0]) * h_ref[...] + aa_ref[...] + ab_ref[...]
    t = jnp.maximum(_dot(z, w1[...]) + b1[...], 0.0)
    o_ref[...] = jnp.maximum(_dot(t, w2[...]) + b2[...], 0.0)


def _gin_ref(h, a_a, a_b, lay, din):
    return pl.pallas_call(
        _gin_ref_body,
        grid=(NPAD // RB,),
        in_specs=[_rows((RB, din))] * 3
        + [_vm((1, 1)), _vm((1, H)), _vm((din, H)), _vm((1, H)), _vm((H, H))],
        out_specs=_rows((RB, H)),
        out_shape=jax.ShapeDtypeStruct((NPAD, H), f32),
    )(h, a_a, a_b, lay["eps"].reshape(1, 1), lay["b1"][None], lay["W1"],
      lay["b2"][None], lay["W2"])


def _gin_ref_final_body(h_ref, aa_ref, ab_ref, eps, b1, w1, b2, w2, bat_ref,
                        z_ref, spool_ref, cnt_ref, ho_ref):
    i = pl.program_id(0)
    z = (1.0 + eps[0, 0]) * h_ref[...] + aa_ref[...] + ab_ref[...]
    t = jnp.maximum(_dot(z, w1[...]) + b1[...], 0.0)
    h = jnp.maximum(_dot(t, w2[...]) + b2[...], 0.0)
    z_ref[...] = h
    _pool_update(i, h, bat_ref, spool_ref, cnt_ref)
    @pl.when(i == pl.num_programs(0) - 1)
    def _():
        ho_ref[...] = spool_ref[...] / jnp.maximum(cnt_ref[...][:, 0:1], 1.0)


def _gin_ref_final(h, a_a, a_b, lay, bat):
    return pl.pallas_call(
        _gin_ref_final_body,
        grid=(NPAD // RB,),
        in_specs=[_rows((RB, H))] * 3
        + [_vm((1, 1)), _vm((1, H)), _vm((H, H)), _vm((1, H)), _vm((H, H)),
           _rows((RB, 1))],
        out_specs=[_rows((RB, H)), _vm((G, H)), _vm((G, W1C)), _vm((G, H))],
        out_shape=[jax.ShapeDtypeStruct((NPAD, H), f32),
                   jax.ShapeDtypeStruct((G, H), f32),
                   jax.ShapeDtypeStruct((G, W1C), f32),
                   jax.ShapeDtypeStruct((G, H), f32)],
    )(h, a_a, a_b, lay["eps"].reshape(1, 1), lay["b1"][None], lay["W1"],
      lay["b2"][None], lay["W2"], bat)


def _gin_first_body(xw_ref, ca_ref, cb_ref, aa_ref, ab_ref, eps, b1, w2, b2,
                    w1n, o_ref, nm_ref):
    nw = ((ca_ref[...] + cb_ref[...])[:, 0:1] > 0.0).astype(f32)
    nm_ref[...] = jnp.broadcast_to(nw, (RB, W1C))
    h = _mlp(xw_ref[...] * nw, aa_ref[...], ab_ref[...], eps, b1, w2, b2)
    ridx = pl.program_id(0) * RB + lax.broadcasted_iota(i32, (RB, 1), 0)
    o_ref[...] = jnp.where(ridx < N, _dot(h, w1n[...]), 0.0)


def _gin_first(xw, c_a, c_b, a_a, a_b, lay, w1n):
    return pl.pallas_call(
        _gin_first_body,
        grid=(NPAD // RB,),
        in_specs=[_rows((RB, H)), _rows((RB, W1C)), _rows((RB, W1C)),
                  _rows((RB, H)), _rows((RB, H)),
                  _vm((1, 1)), _vm((1, H)), _vm((H, H)), _vm((1, H)),
                  _vm((H, H))],
        out_specs=[_rows((RB, H)), _rows((RB, W1C))],
        out_shape=[jax.ShapeDtypeStruct((NPAD, H), f32),
                   jax.ShapeDtypeStruct((NPAD, W1C), f32)],
    )(xw, c_a, c_b, a_a, a_b, lay["eps"].reshape(1, 1), lay["b1"][None],
      lay["W2"], lay["b2"][None], w1n)


def _pool_update(i, h, bat_ref, spool_ref, cnt_ref):
    one_hot = (bat_ref[...] == lax.broadcasted_iota(i32, (1, G), 1)).astype(f32)
    @pl.when(i == 0)
    def _():
        spool_ref[...] = jnp.zeros_like(spool_ref)
        cnt_ref[...] = jnp.zeros_like(cnt_ref)
    spool_ref[...] += lax.dot_general(one_hot, h, (((0,), (0,)), ((), ())),
                                      preferred_element_type=f32,
                                      precision=lax.Precision.HIGHEST)
    cnt_ref[...] += lax.dot_general(one_hot, jnp.ones((RB, W1C), f32),
                                    (((0,), (0,)), ((), ())),
                                    preferred_element_type=f32,
                                    precision=lax.Precision.HIGHEST)


def _gin_final_cls_body(p_ref, aa_ref, ab_ref, eps, b1, w2, b2, bat_ref,
                        cnt_in, wc1, bc1, wc2, bc2,
                        spool_ref, cnt_ref, hs_ref, lg_ref):
    i = pl.program_id(0)
    h = _mlp(p_ref[...], aa_ref[...], ab_ref[...], eps, b1, w2, b2)
    _pool_update(i, h, bat_ref, spool_ref, cnt_ref)
    @pl.when(i == pl.num_programs(0) - 1)
    def _():
        hs = spool_ref[...] / jnp.maximum(cnt_in[...][:, 0:1], 1.0)
        hs_ref[...] = hs
        u = jnp.maximum(_dot(hs, wc1[...]) + bc1[...], 0.0)
        lg_ref[...] = _dot(u, wc2[...]) + bc2[...]


def _gin_final_cls(p, a_a, a_b, lay, bat, cnt8, ep):
    wc2 = jnp.zeros((H, G), f32).at[:, :C].set(ep["Wc2"])
    bc2 = jnp.zeros((1, G), f32).at[0, :C].set(ep["bc2"])
    return pl.pallas_call(
        _gin_final_cls_body,
        grid=(NPAD // RB,),
        in_specs=[_rows((RB, H))] * 3
        + [_vm((1, 1)), _vm((1, H)), _vm((H, H)), _vm((1, H)),
           _rows((RB, 1)), _vm((G, W1C)), _vm((H, H)), _vm((1, H)),
           _vm((H, G)), _vm((1, G))],
        out_specs=[_vm((G, H)), _vm((G, W1C)), _vm((G, H)), _vm((G, G))],
        out_shape=[jax.ShapeDtypeStruct((G, H), f32),
                   jax.ShapeDtypeStruct((G, W1C), f32),
                   jax.ShapeDtypeStruct((G, H), f32),
                   jax.ShapeDtypeStruct((G, G), f32)],
    )(p, a_a, a_b, lay["eps"].reshape(1, 1), lay["b1"][None], lay["W2"],
      lay["b2"][None], bat, cnt8, ep["Wc1"], ep["bc1"][None], wc2, bc2)


def _mask_body(zs_ref, zd_ref, g_ref, sa_ref, da_ref, wma, wmb, bm1, wm2, bm2,
               em_ref, sm_ref, dm_ref):
    u = jnp.maximum(_dot(zs_ref[...], wma[...]) + _dot(zd_ref[...], wmb[...])
                    + bm1[...], 0.0)
    ml = _dot(u, wm2[...]) + bm2[...]
    on = (ml + g_ref[...]) > 0.0
    em_ref[...] = on.astype(f32)
    sm_ref[...] = jnp.where(on, sa_ref[...], jnp.int32(ND))
    dm_ref[...] = jnp.where(on, da_ref[...], jnp.int32(ND))


def _mask(gout, sa2, da2, gpad, wma, wmb, bm1, wm2, bm2):
    nblk = NWCAP // EB
    return pl.pallas_call(
        _mask_body,
        grid=(nblk,),
        in_specs=[pl.BlockSpec((EB, H), lambda j: (j, 0)),
                  pl.BlockSpec((EB, H), lambda j: (j + nblk, 0)),
                  _rows((EB, K)), _rows((EB, 1)), _rows((EB, 1)),
                  _vm((H, K * H)), _vm((H, K * H)), _vm((1, K * H)),
                  _vm((K * H, K)), _vm((1, K))],
        out_specs=[_rows((EB, K))] * 3,
        out_shape=[jax.ShapeDtypeStruct((NWCAP, K), f32),
                   jax.ShapeDtypeStruct((NWCAP, K), i32),
                   jax.ShapeDtypeStruct((NWCAP, K), i32)],
    )(gout, gout, gpad, sa2, da2, wma, wmb, bm1, wm2, bm2)


def kernel(x, params, edge_index, batch):
    src = edge_index[0].astype(i32)
    dst = edge_index[1].astype(i32)

    bkt = dst // BRANGE
    perm = jnp.argsort(bkt, stable=True)
    src_s = src[perm]
    dst_s = dst[perm]
    b_s = bkt[perm]
    counts = jnp.bincount(bkt, length=NW)
    start = jnp.concatenate([jnp.zeros((1,), i32),
                             jnp.cumsum(counts)[:-1].astype(i32)])
    dest = b_s * CAP + jnp.arange(E, dtype=i32) - start[b_s]
    src_b = jnp.full((NWCAP,), ND, i32).at[dest].set(src_s)
    dst_b = jnp.full((NWCAP,), ND, i32).at[dest].set(dst_s)
    src_r = src_b.reshape(NW, NCH_S, CH)
    dst_r = dst_b.reshape(NW, NCH_S, CH)

    x_pad = jnp.pad(x, ((0, NPAD - N), (0, 0)))
    bat = jnp.concatenate([batch.astype(i32),
                           jnp.full((NPAD - N,), -1, i32)])[:, None]
    zrows = jnp.zeros((CH, H), f32)
    zrows8 = jnp.zeros((CH, W1C), f32)
    ones8 = jnp.ones((CH, W1C), f32)

    cz = params["causal"]
    exps = params["experts"]

    wcat = jnp.concatenate(
        [exps[0]["enc"][0]["W1"], exps[1]["enc"][0]["W1"]], axis=1)
    xproj = _t0(x_pad, wcat)

    a0l, a1l = _sc_spmm(x_pad[:, :H], src_r, dst_r, zrows)
    a0r, a1r = _sc_spmm(x_pad[:, H:], src_r, dst_r, zrows)
    a0 = jnp.concatenate([a0l, a0r], axis=1)
    a1 = jnp.concatenate([a1l, a1r], axis=1)
    h1 = _gin_ref(x_pad, a0, a1, cz[0], F)
    a0, a1 = _sc_spmm(h1, src_r, dst_r, zrows)
    h2 = _gin_ref(h1, a0, a1, cz[1], H)
    a0, a1 = _sc_spmm(h2, src_r, dst_r, zrows)
    z_t, _, cnt8, h_orig = _gin_ref_final(h2, a0, a1, cz[2], bat)

    gidx = jnp.concatenate([src_b, dst_b]).reshape(NW, NCH_G, CH)
    gout = _sc_gather_call(z_t, gidx)

    gs = []
    for k in range(K):
        u = jax.random.uniform(jax.random.fold_in(jax.random.key(42), k),
                               (E, 1), dtype=f32)
        gs.append(-jnp.log(-jnp.log(u + 1e-20) + 1e-20))
    g_e = jnp.concatenate(gs, axis=1)
    gpad = jnp.full((NWCAP, K), -1e30, f32).at[dest].set(g_e[perm])

    wma = jnp.concatenate([exps[0]["Wm1"][:H], exps[1]["Wm1"][:H]], axis=1)
    wmb = jnp.concatenate([exps[0]["Wm1"][H:], exps[1]["Wm1"][H:]], axis=1)
    bm1 = jnp.concatenate([exps[0]["bm1"], exps[1]["bm1"]])[None]
    wm2 = jnp.zeros((K * H, K), f32)
    wm2 = wm2.at[:H, 0].set(exps[0]["Wm2"][:, 0]).at[H:, 1].set(
        exps[1]["Wm2"][:, 0])
    bm2 = jnp.concatenate([exps[0]["bm2"], exps[1]["bm2"]])[None]
    em, smap, dmap = _mask(gout, src_b[:, None], dst_b[:, None], gpad,
                           wma, wmb, bm1, wm2, bm2)

    hs_l, lg_l, nm_l = [], [], []
    for k in range(K):
        s_k = smap[:, k]
        d_k = dmap[:, k]
        cidx = jnp.concatenate([s_k, d_k]).reshape(NW, NCH_G, CH)
        cnts = _sc_count_call(cidx, ones8, zrows8)
        sk_r = s_k.reshape(NW, NCH_S, CH)
        enc = exps[k]["enc"]
        xw_k = xproj[:, H * k: H * (k + 1)]
        a0, a1 = _sc_spmm(xw_k, sk_r, dst_r, zrows)
        pe1, nm8 = _gin_first(xw_k, cnts[:NPAD], cnts[NPAD:], a0, a1,
                              enc[0], enc[1]["W1"])
        a0, a1 = _sc_spmm(pe1, sk_r, dst_r, zrows)
        pe2 = _gin_mid(pe1, a0, a1, enc[1], enc[2]["W1"])
        a0, a1 = _sc_spmm(pe2, sk_r, dst_r, zrows)
        _, _, hs_k, lg_k = _gin_final_cls(pe2, a0, a1, enc[2], bat, cnt8,
                                          exps[k])
        hs_l.append(hs_k)
        lg_l.append(lg_k[:, :C])
        nm_l.append(nm8[:N, 0:1])

    expert_logits = jnp.stack(lg_l, axis=1)
    h_stable_st = jnp.stack(hs_l, axis=1)
    node_masks = jnp.stack(nm_l, axis=1)
    slot_by_edge = jnp.zeros((E,), i32).at[perm].set(dest)
    edge_masks = em[slot_by_edge][:, :, None]
    return (expert_logits, h_stable_st, node_masks, edge_masks, h_orig)


# speedup vs baseline: 1.0044x; 1.0044x over previous
import functools

import jax
import jax.numpy as jnp
from jax import lax
from jax.experimental import pallas as pl
from jax.experimental.pallas import tpu as pltpu
from jax.experimental.pallas import tpu_sc as plsc

N = 10000
E = 320000
F = 128
H = 64
G = 128
K = 2
C = 2

NC = 2
NS = 16
NW = NC * NS
CH = 128
NBUF = 4

NPAD = 10240
ND = 10000
BRANGE = NPAD // NW
CAP = 92 * CH
NWCAP = NW * CAP
NCH_S = CAP // CH
NCH_G = 2 * CAP // CH
RPS = NPAD // NS
DUMPL = 320
ACCR = 384
W1C = 8

RB = 2048
EB = 4096

_MESH = plsc.VectorSubcoreMesh(
    core_axis_name="c", subcore_axis_name="s", num_cores=NC, num_subcores=NS)
_SC_PARAMS = pltpu.CompilerParams(use_tc_tiling_on_sc=False)

f32 = jnp.float32
i32 = jnp.int32


def _make_spmm(W):
    def body(table, src_i, dst_i, zrows, out, acc, idx_s, idx_d, rows, *sems):
        cid = lax.axis_index("c")
        sid = lax.axis_index("s")
        wid = sid * NC + cid
        base = sid * RPS
        pltpu.sync_copy(zrows, rows.at[0])
        for j in range(RPS // CH):
            pltpu.sync_copy(rows.at[0], acc.at[pl.ds(base + j * CH, CH)])
        pltpu.sync_copy(src_i.at[wid], idx_s)
        pltpu.sync_copy(dst_i.at[wid], idx_d)
        plsc.subcore_barrier()

        def grp(g, carry):
            cps = [pltpu.async_copy(table.at[idx_s.at[g * NBUF + b]], rows.at[b],
                                    sems[b])
                   for b in range(NBUF)]
            for b in range(NBUF):
                cps[b].wait()
                pltpu.sync_copy(rows.at[b], acc.at[idx_d.at[g * NBUF + b]], add=True)
            return carry

        lax.fori_loop(0, NCH_S // NBUF, grp, 0)
        plsc.subcore_barrier()
        pltpu.sync_copy(acc.at[pl.ds(wid * BRANGE, BRANGE)],
                        out.at[pl.ds(wid * BRANGE, BRANGE)])

    return pl.kernel(
        body,
        out_type=jax.ShapeDtypeStruct((NPAD, W), f32),
        mesh=_MESH,
        scratch_types=[
            pltpu.VMEM_SHARED((NPAD, W), f32),
            pltpu.VMEM((NCH_S, CH), i32),
            pltpu.VMEM((NCH_S, CH), i32),
            pltpu.VMEM((NBUF, CH, W), f32),
        ] + [pltpu.SemaphoreType.DMA] * NBUF,
        compiler_params=_SC_PARAMS,
    )


_sc_spmm = _make_spmm(H)


def _sc_gather_body(table, idx_i, out, idx_v, rows, *sems):
    cid = lax.axis_index("c")
    sid = lax.axis_index("s")
    wid = sid * NC + cid
    pltpu.sync_copy(idx_i.at[wid], idx_v)

    def grp(g, carry):
        cps = [pltpu.async_copy(table.at[idx_v.at[g * NBUF + b]], rows.at[b],
                                sems[b])
               for b in range(NBUF)]
        for b in range(NBUF):
            cps[b].wait()
            r = wid * (NCH_G * CH) + (g * NBUF + b) * CH
            pltpu.sync_copy(rows.at[b], out.at[pl.ds(r, CH)])
        return carry

    lax.fori_loop(0, NCH_G // NBUF, grp, 0)


_sc_gather_call = pl.kernel(
    _sc_gather_body,
    out_type=jax.ShapeDtypeStruct((2 * NWCAP, H), f32),
    mesh=_MESH,
    scratch_types=[
        pltpu.VMEM((NCH_G, CH), i32),
        pltpu.VMEM((NBUF, CH, H), f32),
    ] + [pltpu.SemaphoreType.DMA] * NBUF,
    compiler_params=_SC_PARAMS,
)


def _sc_count_body(idx_i, ones_i, zrows, out, acc, idx_v, ones_v, zbuf):
    cid = lax.axis_index("c")
    sid = lax.axis_index("s")
    wid = sid * NC + cid
    base = sid * RPS
    pltpu.sync_copy(zrows, zbuf)
    for j in range(RPS // CH):
        pltpu.sync_copy(zbuf, acc.at[pl.ds(base + j * CH, CH)])
    pltpu.sync_copy(ones_i, ones_v)
    pltpu.sync_copy(idx_i.at[wid], idx_v)
    plsc.subcore_barrier()

    def step(ch, carry):
        pltpu.sync_copy(ones_v, acc.at[idx_v.at[ch]], add=True)
        return carry

    lax.fori_loop(0, NCH_G, step, 0)
    plsc.subcore_barrier()
    for j in range(RPS // CH):
        r = base + j * CH
        pltpu.sync_copy(acc.at[pl.ds(r, CH)], zbuf)
        pltpu.sync_copy(zbuf, out.at[pl.ds(cid * NPAD + r, CH)])


_sc_count_call = pl.kernel(
    _sc_count_body,
    out_type=jax.ShapeDtypeStruct((NC * NPAD, W1C), f32),
    mesh=_MESH,
    scratch_types=[
        pltpu.VMEM_SHARED((NPAD, W1C), f32),
        pltpu.VMEM((NCH_G, CH), i32),
        pltpu.VMEM((CH, W1C), f32),
        pltpu.VMEM((CH, W1C), f32),
    ],
    compiler_params=_SC_PARAMS,
)


def _vm(shape):
    return pl.BlockSpec(shape, lambda *_: (0,) * len(shape))


def _rows(bshape):
    return pl.BlockSpec(bshape, lambda i: (i,) + (0,) * (len(bshape) - 1))


def _dot(a, b):
    return jnp.dot(a, b, preferred_element_type=f32)


def _t0_body(x_ref, w_ref, o_ref):
    o_ref[...] = _dot(x_ref[...], w_ref[...])


def _t0(x_pad, wcat):
    return pl.pallas_call(
        _t0_body,
        grid=(NPAD // RB,),
        in_specs=[_rows((RB, F)), _vm((F, 2 * H))],
        out_specs=_rows((RB, 2 * H)),
        out_shape=jax.ShapeDtypeStruct((NPAD, 2 * H), f32),
    )(x_pad, wcat)


def _mlp(p, a, eps, b1, w2, b2):
    t = jnp.maximum((1.0 + eps[0, 0]) * p + a + b1[...], 0.0)
    return jnp.maximum(_dot(t, w2[...]) + b2[...], 0.0)


def _gin_mid_body(p_ref, a_ref, eps, b1, w2, b2, w1n, o_ref):
    h = _mlp(p_ref[...], a_ref[...], eps, b1, w2, b2)
    ridx = pl.program_id(0) * RB + lax.broadcasted_iota(i32, (RB, 1), 0)
    o_ref[...] = jnp.where(ridx < N, _dot(h, w1n[...]), 0.0)


def _gin_mid(p, a, lay, w1n):
    return pl.pallas_call(
        _gin_mid_body,
        grid=(NPAD // RB,),
        in_specs=[_rows((RB, H))] * 2
        + [_vm((1, 1)), _vm((1, H)), _vm((H, H)), _vm((1, H)), _vm((H, H))],
        out_specs=_rows((RB, H)),
        out_shape=jax.ShapeDtypeStruct((NPAD, H), f32),
    )(p, a, lay["eps"].reshape(1, 1), lay["b1"][None], lay["W2"],
      lay["b2"][None], w1n)


def _gin_ref_body(h_ref, a_ref, eps, b1, w1, b2, w2, o_ref):
    z = (1.0 + eps[0, 0]) * h_ref[...] + a_ref[...]
    t = jnp.maximum(_dot(z, w1[...]) + b1[...], 0.0)
    o_ref[...] = jnp.maximum(_dot(t, w2[...]) + b2[...], 0.0)


def _gin_ref(h, a, lay, din):
    return pl.pallas_call(
        _gin_ref_body,
        grid=(NPAD // RB,),
        in_specs=[_rows((RB, din))] * 2
        + [_vm((1, 1)), _vm((1, H)), _vm((din, H)), _vm((1, H)), _vm((H, H))],
        out_specs=_rows((RB, H)),
        out_shape=jax.ShapeDtypeStruct((NPAD, H), f32),
    )(h, a, lay["eps"].reshape(1, 1), lay["b1"][None], lay["W1"],
      lay["b2"][None], lay["W2"])


def _gin_ref_final_body(h_ref, a_ref, eps, b1, w1, b2, w2, bat_ref,
                        z_ref, spool_ref, cnt_ref, ho_ref):
    i = pl.program_id(0)
    z = (1.0 + eps[0, 0]) * h_ref[...] + a_ref[...]
    t = jnp.maximum(_dot(z, w1[...]) + b1[...], 0.0)
    h = jnp.maximum(_dot(t, w2[...]) + b2[...], 0.0)
    z_ref[...] = h
    _pool_update(i, h, bat_ref, spool_ref, cnt_ref)
    @pl.when(i == pl.num_programs(0) - 1)
    def _():
        ho_ref[...] = spool_ref[...] / jnp.maximum(cnt_ref[...][:, 0:1], 1.0)


def _gin_ref_final(h, a, lay, bat):
    return pl.pallas_call(
        _gin_ref_final_body,
        grid=(NPAD // RB,),
        in_specs=[_rows((RB, H))] * 2
        + [_vm((1, 1)), _vm((1, H)), _vm((H, H)), _vm((1, H)), _vm((H, H)),
           _rows((RB, 1))],
        out_specs=[_rows((RB, H)), _vm((G, H)), _vm((G, W1C)), _vm((G, H))],
        out_shape=[jax.ShapeDtypeStruct((NPAD, H), f32),
                   jax.ShapeDtypeStruct((G, H), f32),
                   jax.ShapeDtypeStruct((G, W1C), f32),
                   jax.ShapeDtypeStruct((G, H), f32)],
    )(h, a, lay["eps"].reshape(1, 1), lay["b1"][None], lay["W1"],
      lay["b2"][None], lay["W2"], bat)


def _gin_first_body(xw_ref, ca_ref, cb_ref, a_ref, eps, b1, w2, b2,
                    w1n, o_ref, nm_ref):
    nw = ((ca_ref[...] + cb_ref[...])[:, 0:1] > 0.0).astype(f32)
    nm_ref[...] = jnp.broadcast_to(nw, (RB, W1C))
    h = _mlp(xw_ref[...] * nw, a_ref[...], eps, b1, w2, b2)
    ridx = pl.program_id(0) * RB + lax.broadcasted_iota(i32, (RB, 1), 0)
    o_ref[...] = jnp.where(ridx < N, _dot(h, w1n[...]), 0.0)


def _gin_first(xw, c_a, c_b, a, lay, w1n):
    return pl.pallas_call(
        _gin_first_body,
        grid=(NPAD // RB,),
        in_specs=[_rows((RB, H)), _rows((RB, W1C)), _rows((RB, W1C)),
                  _rows((RB, H)),
                  _vm((1, 1)), _vm((1, H)), _vm((H, H)), _vm((1, H)),
                  _vm((H, H))],
        out_specs=[_rows((RB, H)), _rows((RB, W1C))],
        out_shape=[jax.ShapeDtypeStruct((NPAD, H), f32),
                   jax.ShapeDtypeStruct((NPAD, W1C), f32)],
    )(xw, c_a, c_b, a, lay["eps"].reshape(1, 1), lay["b1"][None],
      lay["W2"], lay["b2"][None], w1n)


def _pool_update(i, h, bat_ref, spool_ref, cnt_ref):
    one_hot = (bat_ref[...] == lax.broadcasted_iota(i32, (1, G), 1)).astype(f32)
    @pl.when(i == 0)
    def _():
        spool_ref[...] = jnp.zeros_like(spool_ref)
        cnt_ref[...] = jnp.zeros_like(cnt_ref)
    spool_ref[...] += lax.dot_general(one_hot, h, (((0,), (0,)), ((), ())),
                                      preferred_element_type=f32,
                                      precision=lax.Precision.HIGHEST)
    cnt_ref[...] += lax.dot_general(one_hot, jnp.ones((RB, W1C), f32),
                                    (((0,), (0,)), ((), ())),
                                    preferred_element_type=f32,
                                    precision=lax.Precision.HIGHEST)


def _gin_final_cls_body(p_ref, a_ref, eps, b1, w2, b2, bat_ref,
                        cnt_in, wc1, bc1, wc2, bc2,
                        spool_ref, cnt_ref, hs_ref, lg_ref):
    i = pl.program_id(0)
    h = _mlp(p_ref[...], a_ref[...], eps, b1, w2, b2)
    _pool_update(i, h, bat_ref, spool_ref, cnt_ref)
    @pl.when(i == pl.num_programs(0) - 1)
    def _():
        hs = spool_ref[...] / jnp.maximum(cnt_in[...][:, 0:1], 1.0)
        hs_ref[...] = hs
        u = jnp.maximum(_dot(hs, wc1[...]) + bc1[...], 0.0)
        lg_ref[...] = _dot(u, wc2[...]) + bc2[...]


def _gin_final_cls(p, a, lay, bat, cnt8, ep):
    wc2 = jnp.zeros((H, G), f32).at[:, :C].set(ep["Wc2"])
    bc2 = jnp.zeros((1, G), f32).at[0, :C].set(ep["bc2"])
    return pl.pallas_call(
        _gin_final_cls_body,
        grid=(NPAD // RB,),
        in_specs=[_rows((RB, H))] * 2
        + [_vm((1, 1)), _vm((1, H)), _vm((H, H)), _vm((1, H)),
           _rows((RB, 1)), _vm((G, W1C)), _vm((H, H)), _vm((1, H)),
           _vm((H, G)), _vm((1, G))],
        out_specs=[_vm((G, H)), _vm((G, W1C)), _vm((G, H)), _vm((G, G))],
        out_shape=[jax.ShapeDtypeStruct((G, H), f32),
                   jax.ShapeDtypeStruct((G, W1C), f32),
                   jax.ShapeDtypeStruct((G, H), f32),
                   jax.ShapeDtypeStruct((G, G), f32)],
    )(p, a, lay["eps"].reshape(1, 1), lay["b1"][None], lay["W2"],
      lay["b2"][None], bat, cnt8, ep["Wc1"], ep["bc1"][None], wc2, bc2)


def _mask_body(zs_ref, zd_ref, g_ref, sa_ref, da_ref, wma, wmb, bm1, wm2, bm2,
               em_ref, sm_ref, dm_ref):
    u = jnp.maximum(_dot(zs_ref[...], wma[...]) + _dot(zd_ref[...], wmb[...])
                    + bm1[...], 0.0)
    ml = _dot(u, wm2[...]) + bm2[...]
    on = (ml + g_ref[...]) > 0.0
    em_ref[...] = on.astype(f32)
    sm_ref[...] = jnp.where(on, sa_ref[...], jnp.int32(ND))
    dm_ref[...] = jnp.where(on, da_ref[...], jnp.int32(ND))


def _mask(gout, sa2, da2, gpad, wma, wmb, bm1, wm2, bm2):
    nblk = NWCAP // EB
    return pl.pallas_call(
        _mask_body,
        grid=(nblk,),
        in_specs=[pl.BlockSpec((EB, H), lambda j: (j, 0)),
                  pl.BlockSpec((EB, H), lambda j: (j + nblk, 0)),
                  _rows((EB, K)), _rows((EB, 1)), _rows((EB, 1)),
                  _vm((H, K * H)), _vm((H, K * H)), _vm((1, K * H)),
                  _vm((K * H, K)), _vm((1, K))],
        out_specs=[_rows((EB, K))] * 3,
        out_shape=[jax.ShapeDtypeStruct((NWCAP, K), f32),
                   jax.ShapeDtypeStruct((NWCAP, K), i32),
                   jax.ShapeDtypeStruct((NWCAP, K), i32)],
    )(gout, gout, gpad, sa2, da2, wma, wmb, bm1, wm2, bm2)


def kernel(x, params, edge_index, batch):
    src = edge_index[0].astype(i32)
    dst = edge_index[1].astype(i32)

    bkt = dst // BRANGE
    perm = jnp.argsort(bkt, stable=True)
    src_s = src[perm]
    dst_s = dst[perm]
    b_s = bkt[perm]
    counts = jnp.bincount(bkt, length=NW)
    start = jnp.concatenate([jnp.zeros((1,), i32),
                             jnp.cumsum(counts)[:-1].astype(i32)])
    dest = b_s * CAP + jnp.arange(E, dtype=i32) - start[b_s]
    src_b = jnp.full((NWCAP,), ND, i32).at[dest].set(src_s)
    dst_b = jnp.full((NWCAP,), ND, i32).at[dest].set(dst_s)
    src_r = src_b.reshape(NW, NCH_S, CH)
    dstl_r = dst_b.reshape(NW, NCH_S, CH)

    x_pad = jnp.pad(x, ((0, NPAD - N), (0, 0)))
    bat = jnp.concatenate([batch.astype(i32),
                           jnp.full((NPAD - N,), -1, i32)])[:, None]
    zrows = jnp.zeros((CH, H), f32)
    zrows8 = jnp.zeros((CH, W1C), f32)
    ones8 = jnp.ones((CH, W1C), f32)

    cz = params["causal"]
    exps = params["experts"]

    wcat = jnp.concatenate(
        [exps[0]["enc"][0]["W1"], exps[1]["enc"][0]["W1"]], axis=1)
    xproj = _t0(x_pad, wcat)

    a_l = _sc_spmm(x_pad[:, :H], src_r, dstl_r, zrows)
    a_r = _sc_spmm(x_pad[:, H:], src_r, dstl_r, zrows)
    a = jnp.concatenate([a_l, a_r], axis=1)
    h1 = _gin_ref(x_pad, a, cz[0], F)
    a = _sc_spmm(h1, src_r, dstl_r, zrows)
    h2 = _gin_ref(h1, a, cz[1], H)
    a = _sc_spmm(h2, src_r, dstl_r, zrows)
    z_t, _, cnt8, h_orig = _gin_ref_final(h2, a, cz[2], bat)

    gidx = jnp.concatenate([src_b, dst_b]).reshape(NW, NCH_G, CH)
    gout = _sc_gather_call(z_t, gidx)

    gs = []
    for k in range(K):
        u = jax.random.uniform(jax.random.fold_in(jax.random.key(42), k),
                               (E, 1), dtype=f32)
        gs.append(-jnp.log(-jnp.log(u + 1e-20) + 1e-20))
    g_e = jnp.concatenate(gs, axis=1)
    gpad = jnp.full((NWCAP, K), -1e30, f32).at[dest].set(g_e[perm])

    wma = jnp.concatenate([exps[0]["Wm1"][:H], exps[1]["Wm1"][:H]], axis=1)
    wmb = jnp.concatenate([exps[0]["Wm1"][H:], exps[1]["Wm1"][H:]], axis=1)
    bm1 = jnp.concatenate([exps[0]["bm1"], exps[1]["bm1"]])[None]
    wm2 = jnp.zeros((K * H, K), f32)
    wm2 = wm2.at[:H, 0].set(exps[0]["Wm2"][:, 0]).at[H:, 1].set(
        exps[1]["Wm2"][:, 0])
    bm2 = jnp.concatenate([exps[0]["bm2"], exps[1]["bm2"]])[None]
    em, smap, dmap = _mask(gout, src_b[:, None], dst_b[:, None], gpad,
                           wma, wmb, bm1, wm2, bm2)

    hs_l, lg_l, nm_l = [], [], []
    for k in range(K):
        s_k = smap[:, k]
        d_k = dmap[:, k]
        cidx = jnp.concatenate([s_k, d_k]).reshape(NW, NCH_G, CH)
        cnts = _sc_count_call(cidx, ones8, zrows8)
        sk_r = s_k.reshape(NW, NCH_S, CH)
        enc = exps[k]["enc"]
        xw_k = xproj[:, H * k: H * (k + 1)]
        a = _sc_spmm(xw_k, sk_r, dstl_r, zrows)
        pe1, nm8 = _gin_first(xw_k, cnts[:NPAD], cnts[NPAD:], a,
                              enc[0], enc[1]["W1"])
        a = _sc_spmm(pe1, sk_r, dstl_r, zrows)
        pe2 = _gin_mid(pe1, a, enc[1], enc[2]["W1"])
        a = _sc_spmm(pe2, sk_r, dstl_r, zrows)
        _, _, hs_k, lg_k = _gin_final_cls(pe2, a, enc[2], bat, cnt8,
                                          exps[k])
        hs_l.append(hs_k)
        lg_l.append(lg_k[:, :C])
        nm_l.append(nm8[:N, 0:1])

    expert_logits = jnp.stack(lg_l, axis=1)
    h_stable_st = jnp.stack(hs_l, axis=1)
    node_masks = jnp.stack(nm_l, axis=1)
    slot_by_edge = jnp.zeros((E,), i32).at[perm].set(dest)
    edge_masks = em[slot_by_edge][:, :, None]
    return (expert_logits, h_stable_st, node_masks, edge_masks, h_orig)
